# baseline (speedup 1.0000x reference)
import functools

import jax
import jax.numpy as jnp
from jax import lax
from jax.experimental import pallas as pl
from jax.experimental.pallas import tpu as pltpu
from jax.experimental.pallas import tpu_sc as plsc

N = 10000
E = 320000
T_PAD = 10008

NC = 2
NS = 16

CW = 128
E_PAD = 327680
EROWS = E_PAD // CW
N_EXTRA = E_PAD - E
ROWS_PER_W = EROWS // (NC * NS)
CH = 8
HB = 8
DP = 64
N_ACC = 10112
NT = N_ACC // NS
CHUNKS = (128, 128, 128, 128, 120)

_mesh = plsc.VectorSubcoreMesh(core_axis_name="c", subcore_axis_name="s")


def _zero_vmem_2d(ref, nrows, ncols):
    per_row = ncols // 16

    def body(i, _):
        r = i // per_row
        col = (i % per_row) * 16
        ref[r, pl.ds(col, 16)] = jnp.zeros((16,), jnp.float32)
        return 0

    lax.fori_loop(0, nrows * per_row, body, 0)


def _agg_body(*refs, d, npass):
    t_hbms = refs[:npass]
    src_hbm, dst_hbm = refs[npass:npass + 2]
    out_hbms = refs[npass + 2:2 * npass + 2]
    src_v, dst_v, rows_v, zero_v, acc_sh, sem, sem2 = refs[2 * npass + 2:]
    c = lax.axis_index("c")
    s = lax.axis_index("s")
    w = c * NS + s
    row0 = w * ROWS_PER_W

    _zero_vmem_2d(zero_v, CW, d)

    for p in range(npass):
        off = 0
        for sz in CHUNKS:
            pltpu.sync_copy(zero_v.at[pl.ds(0, sz)],
                            acc_sh.at[pl.ds(s * NT + off, sz)])
            off += sz
        plsc.subcore_barrier()

        def body(g, _):
            base = row0 + g * CH
            pltpu.sync_copy(src_hbm.at[pl.ds(base, CH)], src_v)
            pltpu.sync_copy(dst_hbm.at[pl.ds(base, CH)], dst_v)
            for half in range(CH // HB):
                descs = [pltpu.async_copy(t_hbms[p].at[src_v.at[half * HB + j]],
                                          rows_v.at[j], sem)
                         for j in range(HB)]
                for dsc in descs:
                    dsc.wait()
                sdescs = [pltpu.async_copy(rows_v.at[j],
                                           acc_sh.at[dst_v.at[half * HB + j]],
                                           sem2, add=True)
                          for j in range(HB)]
                for dsc in sdescs:
                    dsc.wait()
            return 0

        lax.fori_loop(0, ROWS_PER_W // CH, body, 0)
        plsc.subcore_barrier()

        off = 0
        for sz in CHUNKS:
            b = s * NT + off
            pltpu.sync_copy(acc_sh.at[pl.ds(b, sz)], rows_v.at[0, pl.ds(0, sz)])
            pltpu.sync_copy(rows_v.at[0, pl.ds(0, sz)],
                            out_hbms[p].at[c, pl.ds(b, sz)])
            off += sz


@functools.lru_cache(maxsize=None)
def _make_scatter(npass):
    d = DP
    return pl.kernel(
        functools.partial(_agg_body, d=d, npass=npass),
        out_type=[jax.ShapeDtypeStruct((NC, N_ACC, d), jnp.float32)
                  for _ in range(npass)],
        mesh=_mesh,
        compiler_params=pltpu.CompilerParams(use_tc_tiling_on_sc=False),
        scratch_types=[
            pltpu.VMEM((CH, CW), jnp.int32),
            pltpu.VMEM((CH, CW), jnp.int32),
            pltpu.VMEM((HB, CW, d), jnp.float32),
            pltpu.VMEM((CW, d), jnp.float32),
            pltpu.VMEM_SHARED((N_ACC, d), jnp.float32),
            pltpu.SemaphoreType.DMA,
            pltpu.SemaphoreType.DMA,
        ],
    )


BN = 1000


def _mm_body(x_ref, w_ref, o_ref):
    o_ref[...] = jnp.dot(x_ref[...], w_ref[...],
                         preferred_element_type=jnp.float32)


def _tc_mm(x, w):
    n, k = x.shape
    d = w.shape[1]
    return pl.pallas_call(
        _mm_body,
        grid=(n // BN,),
        in_specs=[
            pl.BlockSpec((BN, k), lambda i: (i, 0)),
            pl.BlockSpec((k, d), lambda i: (0, 0)),
        ],
        out_specs=pl.BlockSpec((BN, d), lambda i: (i, 0)),
        out_shape=jax.ShapeDtypeStruct((n, d), jnp.float32),
    )(x, w)


def _fused_body(*refs, relu, np_):
    p_refs = refs[:np_]
    invd_ref, x_ref, w_ref, b_ref, o_ref = refs[np_:]
    agg = jnp.concatenate([pr[0] + pr[1] for pr in p_refs], axis=1)
    r = agg * invd_ref[...] + jnp.dot(x_ref[...], w_ref[...],
                                      preferred_element_type=jnp.float32) + b_ref[...]
    o_ref[...] = jnp.maximum(r, 0.0) if relu else r


def _tc_fused(panels, invd, x, w, b2d, relu):
    n, k = x.shape
    d = w.shape[1]
    np_ = len(panels)
    return pl.pallas_call(
        functools.partial(_fused_body, relu=relu, np_=np_),
        grid=(n // BN,),
        in_specs=[pl.BlockSpec((NC, BN, DP), lambda i: (0, i, 0))
                  for _ in range(np_)] + [
            pl.BlockSpec((BN, 1), lambda i: (i, 0)),
            pl.BlockSpec((BN, k), lambda i: (i, 0)),
            pl.BlockSpec((k, d), lambda i: (0, 0)),
            pl.BlockSpec((1, d), lambda i: (0, 0)),
        ],
        out_specs=pl.BlockSpec((BN, d), lambda i: (i, 0)),
        out_shape=jax.ShapeDtypeStruct((n, d), jnp.float32),
    )(*panels, invd, x, w, b2d)


def _invd_body(dp_ref, o_ref):
    deg = dp_ref[0, :, 0:1] + dp_ref[1, :, 0:1]
    o_ref[...] = 1.0 / jnp.maximum(deg, 1.0)


def _tc_invd(degp):
    return pl.pallas_call(
        _invd_body,
        grid=(N // BN,),
        in_specs=[pl.BlockSpec((NC, BN, DP), lambda i: (0, i, 0))],
        out_specs=pl.BlockSpec((BN, 1), lambda i: (i, 0)),
        out_shape=jax.ShapeDtypeStruct((N, 1), jnp.float32),
    )(degp)


def kernel(x, edge_index, Wl1, Wr1, b1, Wl2, Wr2, b2, Wl3, Wr3, b3):
    src = edge_index[0]
    dst = edge_index[1]
    src2 = jnp.concatenate([src, jnp.full((N_EXTRA,), N, jnp.int32)]).reshape(EROWS, CW)
    pad_dst = N + (jnp.arange(N_EXTRA, dtype=jnp.int32) % (N_ACC - N))
    dst2 = jnp.concatenate([dst, pad_dst]).reshape(EROWS, CW)

    ones_mat = jnp.concatenate([jnp.ones((N, DP), jnp.float32),
                                jnp.zeros((T_PAD - N, DP), jnp.float32)])
    zpad = jnp.zeros((T_PAD - N, DP), jnp.float32)

    def t_panels(h, wl, d):
        t = _tc_mm(h, wl)
        return [jnp.concatenate([t[:, j * DP:(j + 1) * DP], zpad], axis=0)
                for j in range(d // DP)]

    p1lo, p1hi, degp = _make_scatter(3)(*t_panels(x, Wl1, 128), ones_mat,
                                        src2, dst2)
    invd = _tc_invd(degp)
    h1 = _tc_fused([p1lo, p1hi], invd, x, Wr1, b1.reshape(1, 128), True)

    p2lo, p2hi = _make_scatter(2)(*t_panels(h1, Wl2, 128), src2, dst2)
    h2 = _tc_fused([p2lo, p2hi], invd, h1, Wr2, b2.reshape(1, 128), True)

    (p3,) = _make_scatter(1)(*t_panels(h2, Wl3, 64), src2, dst2)
    return _tc_fused([p3], invd, h2, Wr3, b3.reshape(1, 64), False)


# speedup vs baseline: 1.7065x; 1.7065x over previous
import functools

import jax
import jax.numpy as jnp
from jax import lax
from jax.experimental import pallas as pl
from jax.experimental.pallas import tpu as pltpu
from jax.experimental.pallas import tpu_sc as plsc

N = 10000
E = 320000
T_PAD = 10008

NC = 2
NS = 16

CW = 128
E_PAD = 327680
EROWS = E_PAD // CW
N_EXTRA = E_PAD - E
ROWS_PER_W = EROWS // (NC * NS)
CH = 8
HB = 8
DP = 64
N_ACC = 10112
NT = N_ACC // NS
CHUNKS = (128, 128, 128, 128, 120)

_mesh = plsc.VectorSubcoreMesh(core_axis_name="c", subcore_axis_name="s")


def _zero_vmem_2d(ref, nrows, ncols):
    per_row = ncols // 32

    def body(i, _):
        r = i // per_row
        col = (i % per_row) * 32
        ref[r, pl.ds(col, 32)] = jnp.zeros((32,), jnp.bfloat16)
        return 0

    lax.fori_loop(0, nrows * per_row, body, 0)


def _agg_body(*refs, d, npass):
    t_hbms = refs[:npass]
    src_hbm, dst_hbm = refs[npass:npass + 2]
    out_hbms = refs[npass + 2:2 * npass + 2]
    src_v, dst_v, rows_v, zero_v, acc_sh, sem, sem2 = refs[2 * npass + 2:]
    c = lax.axis_index("c")
    s = lax.axis_index("s")
    w = c * NS + s
    row0 = w * ROWS_PER_W

    _zero_vmem_2d(zero_v, CW, d)

    for p in range(npass):
        off = 0
        for sz in CHUNKS:
            pltpu.sync_copy(zero_v.at[pl.ds(0, sz)],
                            acc_sh.at[pl.ds(s * NT + off, sz)])
            off += sz
        plsc.subcore_barrier()

        def body(g, _):
            base = row0 + g * CH
            pltpu.sync_copy(src_hbm.at[pl.ds(base, CH)], src_v)
            pltpu.sync_copy(dst_hbm.at[pl.ds(base, CH)], dst_v)
            for half in range(CH // HB):
                descs = [pltpu.async_copy(t_hbms[p].at[src_v.at[half * HB + j]],
                                          rows_v.at[j], sem)
                         for j in range(HB)]
                for dsc in descs:
                    dsc.wait()
                sdescs = [pltpu.async_copy(rows_v.at[j],
                                           acc_sh.at[dst_v.at[half * HB + j]],
                                           sem2, add=True)
                          for j in range(HB)]
                for dsc in sdescs:
                    dsc.wait()
            return 0

        lax.fori_loop(0, ROWS_PER_W // CH, body, 0)
        plsc.subcore_barrier()

        off = 0
        for sz in CHUNKS:
            b = s * NT + off
            pltpu.sync_copy(acc_sh.at[pl.ds(b, sz)], rows_v.at[0, pl.ds(0, sz)])
            pltpu.sync_copy(rows_v.at[0, pl.ds(0, sz)],
                            out_hbms[p].at[c, pl.ds(b, sz)])
            off += sz


@functools.lru_cache(maxsize=None)
def _make_scatter(npass):
    d = DP
    return pl.kernel(
        functools.partial(_agg_body, d=d, npass=npass),
        out_type=[jax.ShapeDtypeStruct((NC, N_ACC, d), jnp.bfloat16)
                  for _ in range(npass)],
        mesh=_mesh,
        compiler_params=pltpu.CompilerParams(use_tc_tiling_on_sc=False),
        scratch_types=[
            pltpu.VMEM((CH, CW), jnp.int32),
            pltpu.VMEM((CH, CW), jnp.int32),
            pltpu.VMEM((HB, CW, d), jnp.bfloat16),
            pltpu.VMEM((CW, d), jnp.bfloat16),
            pltpu.VMEM_SHARED((N_ACC, d), jnp.bfloat16),
            pltpu.SemaphoreType.DMA,
            pltpu.SemaphoreType.DMA,
        ],
    )


BN = 1000


def _mm_body(x_ref, w_ref, o_ref):
    o_ref[...] = jnp.dot(x_ref[...], w_ref[...],
                         preferred_element_type=jnp.float32).astype(jnp.bfloat16)


def _tc_mm(x, w):
    n, k = x.shape
    d = w.shape[1]
    return pl.pallas_call(
        _mm_body,
        grid=(n // BN,),
        in_specs=[
            pl.BlockSpec((BN, k), lambda i: (i, 0)),
            pl.BlockSpec((k, d), lambda i: (0, 0)),
        ],
        out_specs=pl.BlockSpec((BN, d), lambda i: (i, 0)),
        out_shape=jax.ShapeDtypeStruct((n, d), jnp.bfloat16),
    )(x, w)


def _fused_body(*refs, relu, np_):
    p_refs = refs[:np_]
    invd_ref, x_ref, w_ref, b_ref, o_ref = refs[np_:]
    agg = jnp.concatenate(
        [pr[0].astype(jnp.float32) + pr[1].astype(jnp.float32)
         for pr in p_refs], axis=1)
    r = agg * invd_ref[...] + jnp.dot(x_ref[...], w_ref[...],
                                      preferred_element_type=jnp.float32) + b_ref[...]
    o_ref[...] = jnp.maximum(r, 0.0) if relu else r


def _tc_fused(panels, invd, x, w, b2d, relu):
    n, k = x.shape
    d = w.shape[1]
    np_ = len(panels)
    return pl.pallas_call(
        functools.partial(_fused_body, relu=relu, np_=np_),
        grid=(n // BN,),
        in_specs=[pl.BlockSpec((NC, BN, DP), lambda i: (0, i, 0))
                  for _ in range(np_)] + [
            pl.BlockSpec((BN, 1), lambda i: (i, 0)),
            pl.BlockSpec((BN, k), lambda i: (i, 0)),
            pl.BlockSpec((k, d), lambda i: (0, 0)),
            pl.BlockSpec((1, d), lambda i: (0, 0)),
        ],
        out_specs=pl.BlockSpec((BN, d), lambda i: (i, 0)),
        out_shape=jax.ShapeDtypeStruct((n, d), jnp.float32),
    )(*panels, invd, x, w, b2d)


def _invd_body(dp_ref, o_ref):
    deg = (dp_ref[0, :, 0:1].astype(jnp.float32)
           + dp_ref[1, :, 0:1].astype(jnp.float32))
    o_ref[...] = 1.0 / jnp.maximum(deg, 1.0)


def _tc_invd(degp):
    return pl.pallas_call(
        _invd_body,
        grid=(N // BN,),
        in_specs=[pl.BlockSpec((NC, BN, DP), lambda i: (0, i, 0))],
        out_specs=pl.BlockSpec((BN, 1), lambda i: (i, 0)),
        out_shape=jax.ShapeDtypeStruct((N, 1), jnp.float32),
    )(degp)


def kernel(x, edge_index, Wl1, Wr1, b1, Wl2, Wr2, b2, Wl3, Wr3, b3):
    src = edge_index[0]
    dst = edge_index[1]
    src2 = jnp.concatenate([src, jnp.full((N_EXTRA,), N, jnp.int32)]).reshape(EROWS, CW)
    pad_dst = N + (jnp.arange(N_EXTRA, dtype=jnp.int32) % (N_ACC - N))
    dst2 = jnp.concatenate([dst, pad_dst]).reshape(EROWS, CW)

    ones_mat = jnp.concatenate([jnp.ones((N, DP), jnp.bfloat16),
                                jnp.zeros((T_PAD - N, DP), jnp.bfloat16)])
    zpad = jnp.zeros((T_PAD - N, DP), jnp.bfloat16)

    def t_panels(h, wl, d):
        t = _tc_mm(h, wl)
        return [jnp.concatenate([t[:, j * DP:(j + 1) * DP], zpad], axis=0)
                for j in range(d // DP)]

    p1lo, p1hi, degp = _make_scatter(3)(*t_panels(x, Wl1, 128), ones_mat,
                                        src2, dst2)
    invd = _tc_invd(degp)
    h1 = _tc_fused([p1lo, p1hi], invd, x, Wr1, b1.reshape(1, 128), True)

    p2lo, p2hi = _make_scatter(2)(*t_panels(h1, Wl2, 128), src2, dst2)
    h2 = _tc_fused([p2lo, p2hi], invd, h1, Wr2, b2.reshape(1, 128), True)

    (p3,) = _make_scatter(1)(*t_panels(h2, Wl3, 64), src2, dst2)
    return _tc_fused([p3], invd, h2, Wr3, b3.reshape(1, 64), False)


# speedup vs baseline: 1.7145x; 1.0047x over previous
import functools

import jax
import jax.numpy as jnp
from jax import lax
from jax.experimental import pallas as pl
from jax.experimental.pallas import tpu as pltpu
from jax.experimental.pallas import tpu_sc as plsc

N = 10000
E = 320000
T_PAD = 10008

NC = 2
NS = 16

CW = 128
E_PAD = 327680
EROWS = E_PAD // CW
N_EXTRA = E_PAD - E
ROWS_PER_W = EROWS // (NC * NS)
CH = 8
HB = 8
DP = 64
N_ACC = 10112
NT = N_ACC // NS
CHUNKS = (128, 128, 128, 128, 120)

_mesh = plsc.VectorSubcoreMesh(core_axis_name="c", subcore_axis_name="s")


def _zero_vmem_2d(ref, nrows, ncols):
    per_row = ncols // 32

    def body(i, _):
        r = i // per_row
        col = (i % per_row) * 32
        ref[r, pl.ds(col, 32)] = jnp.zeros((32,), jnp.bfloat16)
        return 0

    lax.fori_loop(0, nrows * per_row, body, 0)


def _agg_body(*refs, d, npass):
    t_hbms = refs[:npass]
    src_hbm, dst_hbm = refs[npass:npass + 2]
    out_hbms = refs[npass + 2:2 * npass + 2]
    src_v, dst_v, rows_v, zero_v, acc_sh, sem, sem2 = refs[2 * npass + 2:]
    c = lax.axis_index("c")
    s = lax.axis_index("s")
    w = c * NS + s
    row0 = w * ROWS_PER_W

    _zero_vmem_2d(zero_v, CW, d)

    for p in range(npass):
        zdescs = []
        off = 0
        for sz in CHUNKS:
            zdescs.append(pltpu.async_copy(
                zero_v.at[pl.ds(0, sz)],
                acc_sh.at[pl.ds(s * NT + off, sz)], sem))
            off += sz
        for dsc in zdescs:
            dsc.wait()
        plsc.subcore_barrier()

        def body(g, _):
            base = row0 + g * CH
            pltpu.sync_copy(src_hbm.at[pl.ds(base, CH)], src_v)
            pltpu.sync_copy(dst_hbm.at[pl.ds(base, CH)], dst_v)
            for half in range(CH // HB):
                descs = [pltpu.async_copy(t_hbms[p].at[src_v.at[half * HB + j]],
                                          rows_v.at[j], sem)
                         for j in range(HB)]
                for dsc in descs:
                    dsc.wait()
                sdescs = [pltpu.async_copy(rows_v.at[j],
                                           acc_sh.at[dst_v.at[half * HB + j]],
                                           sem2, add=True)
                          for j in range(HB)]
                for dsc in sdescs:
                    dsc.wait()
            return 0

        lax.fori_loop(0, ROWS_PER_W // CH, body, 0)
        plsc.subcore_barrier()

        rdescs = []
        off = 0
        for k, sz in enumerate(CHUNKS):
            rdescs.append(pltpu.async_copy(
                acc_sh.at[pl.ds(s * NT + off, sz)],
                rows_v.at[k, pl.ds(0, sz)], sem))
            off += sz
        for dsc in rdescs:
            dsc.wait()
        wdescs = []
        off = 0
        for k, sz in enumerate(CHUNKS):
            wdescs.append(pltpu.async_copy(
                rows_v.at[k, pl.ds(0, sz)],
                out_hbms[p].at[c, pl.ds(s * NT + off, sz)], sem2))
            off += sz
        for dsc in wdescs:
            dsc.wait()


@functools.lru_cache(maxsize=None)
def _make_scatter(npass):
    d = DP
    return pl.kernel(
        functools.partial(_agg_body, d=d, npass=npass),
        out_type=[jax.ShapeDtypeStruct((NC, N_ACC, d), jnp.bfloat16)
                  for _ in range(npass)],
        mesh=_mesh,
        compiler_params=pltpu.CompilerParams(use_tc_tiling_on_sc=False),
        scratch_types=[
            pltpu.VMEM((CH, CW), jnp.int32),
            pltpu.VMEM((CH, CW), jnp.int32),
            pltpu.VMEM((HB, CW, d), jnp.bfloat16),
            pltpu.VMEM((CW, d), jnp.bfloat16),
            pltpu.VMEM_SHARED((N_ACC, d), jnp.bfloat16),
            pltpu.SemaphoreType.DMA,
            pltpu.SemaphoreType.DMA,
        ],
    )


BN = 1000


def _mm_body(x_ref, w_ref, o_ref):
    o_ref[...] = jnp.dot(x_ref[...], w_ref[...],
                         preferred_element_type=jnp.float32).astype(jnp.bfloat16)


def _tc_mm(x, w):
    n, k = x.shape
    d = w.shape[1]
    return pl.pallas_call(
        _mm_body,
        grid=(n // BN,),
        in_specs=[
            pl.BlockSpec((BN, k), lambda i: (i, 0)),
            pl.BlockSpec((k, d), lambda i: (0, 0)),
        ],
        out_specs=pl.BlockSpec((BN, d), lambda i: (i, 0)),
        out_shape=jax.ShapeDtypeStruct((n, d), jnp.bfloat16),
    )(x, w)


def _fused_body(*refs, relu, np_):
    p_refs = refs[:np_]
    invd_ref, x_ref, w_ref, b_ref, o_ref = refs[np_:]
    agg = jnp.concatenate(
        [pr[0].astype(jnp.float32) + pr[1].astype(jnp.float32)
         for pr in p_refs], axis=1)
    r = agg * invd_ref[...] + jnp.dot(x_ref[...], w_ref[...],
                                      preferred_element_type=jnp.float32) + b_ref[...]
    o_ref[...] = jnp.maximum(r, 0.0) if relu else r


def _tc_fused(panels, invd, x, w, b2d, relu):
    n, k = x.shape
    d = w.shape[1]
    np_ = len(panels)
    return pl.pallas_call(
        functools.partial(_fused_body, relu=relu, np_=np_),
        grid=(n // BN,),
        in_specs=[pl.BlockSpec((NC, BN, DP), lambda i: (0, i, 0))
                  for _ in range(np_)] + [
            pl.BlockSpec((BN, 1), lambda i: (i, 0)),
            pl.BlockSpec((BN, k), lambda i: (i, 0)),
            pl.BlockSpec((k, d), lambda i: (0, 0)),
            pl.BlockSpec((1, d), lambda i: (0, 0)),
        ],
        out_specs=pl.BlockSpec((BN, d), lambda i: (i, 0)),
        out_shape=jax.ShapeDtypeStruct((n, d), jnp.float32),
    )(*panels, invd, x, w, b2d)


def _invd_body(dp_ref, o_ref):
    deg = (dp_ref[0, :, 0:1].astype(jnp.float32)
           + dp_ref[1, :, 0:1].astype(jnp.float32))
    o_ref[...] = 1.0 / jnp.maximum(deg, 1.0)


def _tc_invd(degp):
    return pl.pallas_call(
        _invd_body,
        grid=(N // BN,),
        in_specs=[pl.BlockSpec((NC, BN, DP), lambda i: (0, i, 0))],
        out_specs=pl.BlockSpec((BN, 1), lambda i: (i, 0)),
        out_shape=jax.ShapeDtypeStruct((N, 1), jnp.float32),
    )(degp)


def kernel(x, edge_index, Wl1, Wr1, b1, Wl2, Wr2, b2, Wl3, Wr3, b3):
    src = edge_index[0]
    dst = edge_index[1]
    src2 = jnp.concatenate([src, jnp.full((N_EXTRA,), N, jnp.int32)]).reshape(EROWS, CW)
    pad_dst = N + (jnp.arange(N_EXTRA, dtype=jnp.int32) % (N_ACC - N))
    dst2 = jnp.concatenate([dst, pad_dst]).reshape(EROWS, CW)

    ones_mat = jnp.concatenate([jnp.ones((N, DP), jnp.bfloat16),
                                jnp.zeros((T_PAD - N, DP), jnp.bfloat16)])
    zpad = jnp.zeros((T_PAD - N, DP), jnp.bfloat16)

    def t_panels(h, wl, d):
        t = _tc_mm(h, wl)
        return [jnp.concatenate([t[:, j * DP:(j + 1) * DP], zpad], axis=0)
                for j in range(d // DP)]

    p1lo, p1hi, degp = _make_scatter(3)(*t_panels(x, Wl1, 128), ones_mat,
                                        src2, dst2)
    invd = _tc_invd(degp)
    h1 = _tc_fused([p1lo, p1hi], invd, x, Wr1, b1.reshape(1, 128), True)

    p2lo, p2hi = _make_scatter(2)(*t_panels(h1, Wl2, 128), src2, dst2)
    h2 = _tc_fused([p2lo, p2hi], invd, h1, Wr2, b2.reshape(1, 128), True)

    (p3,) = _make_scatter(1)(*t_panels(h2, Wl3, 64), src2, dst2)
    return _tc_fused([p3], invd, h2, Wr3, b3.reshape(1, 64), False)


# speedup vs baseline: 1.7581x; 1.0255x over previous
import functools

import jax
import jax.numpy as jnp
from jax import lax
from jax.experimental import pallas as pl
from jax.experimental.pallas import tpu as pltpu
from jax.experimental.pallas import tpu_sc as plsc

N = 10000
E = 320000
T_PAD = 10008

NC = 2
NS = 16

CW = 128
E_PAD = 327680
EROWS = E_PAD // CW
N_EXTRA = E_PAD - E
ROWS_PER_W = EROWS // (NC * NS)
CH = 16
HB = 8
DP = 64
N_ACC = 10112
NT = N_ACC // NS
CHUNKS = (128, 128, 128, 128, 120)

_mesh = plsc.VectorSubcoreMesh(core_axis_name="c", subcore_axis_name="s")


def _zero_vmem_2d(ref, nrows, ncols):
    per_row = ncols // 32

    def body(i, _):
        r = i // per_row
        col = (i % per_row) * 32
        ref[r, pl.ds(col, 32)] = jnp.zeros((32,), jnp.bfloat16)
        return 0

    lax.fori_loop(0, nrows * per_row, body, 0)


def _agg_body(*refs, d, npass):
    t_hbms = refs[:npass]
    src_hbm, dst_hbm = refs[npass:npass + 2]
    out_hbms = refs[npass + 2:2 * npass + 2]
    src_v, dst_v, rows_v, zero_v, acc_sh, sem, sem2 = refs[2 * npass + 2:]
    c = lax.axis_index("c")
    s = lax.axis_index("s")
    w = c * NS + s
    row0 = w * ROWS_PER_W

    _zero_vmem_2d(zero_v, CW, d)

    for p in range(npass):
        zdescs = []
        off = 0
        for sz in CHUNKS:
            zdescs.append(pltpu.async_copy(
                zero_v.at[pl.ds(0, sz)],
                acc_sh.at[pl.ds(s * NT + off, sz)], sem))
            off += sz
        for dsc in zdescs:
            dsc.wait()
        plsc.subcore_barrier()

        def body(g, _):
            base = row0 + g * CH
            pltpu.sync_copy(src_hbm.at[pl.ds(base, CH)], src_v)
            pltpu.sync_copy(dst_hbm.at[pl.ds(base, CH)], dst_v)
            for half in range(CH // HB):
                descs = [pltpu.async_copy(t_hbms[p].at[src_v.at[half * HB + j]],
                                          rows_v.at[j], sem)
                         for j in range(HB)]
                for dsc in descs:
                    dsc.wait()
                sdescs = [pltpu.async_copy(rows_v.at[j],
                                           acc_sh.at[dst_v.at[half * HB + j]],
                                           sem2, add=True)
                          for j in range(HB)]
                for dsc in sdescs:
                    dsc.wait()
            return 0

        lax.fori_loop(0, ROWS_PER_W // CH, body, 0)
        plsc.subcore_barrier()

        rdescs = []
        off = 0
        for k, sz in enumerate(CHUNKS):
            rdescs.append(pltpu.async_copy(
                acc_sh.at[pl.ds(s * NT + off, sz)],
                rows_v.at[k, pl.ds(0, sz)], sem))
            off += sz
        for dsc in rdescs:
            dsc.wait()
        wdescs = []
        off = 0
        for k, sz in enumerate(CHUNKS):
            wdescs.append(pltpu.async_copy(
                rows_v.at[k, pl.ds(0, sz)],
                out_hbms[p].at[c, pl.ds(s * NT + off, sz)], sem2))
            off += sz
        for dsc in wdescs:
            dsc.wait()


@functools.lru_cache(maxsize=None)
def _make_scatter(npass):
    d = DP
    return pl.kernel(
        functools.partial(_agg_body, d=d, npass=npass),
        out_type=[jax.ShapeDtypeStruct((NC, N_ACC, d), jnp.bfloat16)
                  for _ in range(npass)],
        mesh=_mesh,
        compiler_params=pltpu.CompilerParams(use_tc_tiling_on_sc=False),
        scratch_types=[
            pltpu.VMEM((CH, CW), jnp.int32),
            pltpu.VMEM((CH, CW), jnp.int32),
            pltpu.VMEM((HB, CW, d), jnp.bfloat16),
            pltpu.VMEM((CW, d), jnp.bfloat16),
            pltpu.VMEM_SHARED((N_ACC, d), jnp.bfloat16),
            pltpu.SemaphoreType.DMA,
            pltpu.SemaphoreType.DMA,
        ],
    )


BN = 1000


def _mm_body(x_ref, w_ref, o_ref):
    o_ref[...] = jnp.dot(x_ref[...], w_ref[...],
                         preferred_element_type=jnp.float32).astype(jnp.bfloat16)


def _tc_mm(x, w):
    n, k = x.shape
    d = w.shape[1]
    return pl.pallas_call(
        _mm_body,
        grid=(n // BN,),
        in_specs=[
            pl.BlockSpec((BN, k), lambda i: (i, 0)),
            pl.BlockSpec((k, d), lambda i: (0, 0)),
        ],
        out_specs=pl.BlockSpec((BN, d), lambda i: (i, 0)),
        out_shape=jax.ShapeDtypeStruct((n, d), jnp.bfloat16),
    )(x, w)


def _fused_body(*refs, relu, np_):
    p_refs = refs[:np_]
    invd_ref, x_ref, w_ref, b_ref, o_ref = refs[np_:]
    agg = jnp.concatenate(
        [pr[0].astype(jnp.float32) + pr[1].astype(jnp.float32)
         for pr in p_refs], axis=1)
    r = agg * invd_ref[...] + jnp.dot(x_ref[...], w_ref[...],
                                      preferred_element_type=jnp.float32) + b_ref[...]
    o_ref[...] = jnp.maximum(r, 0.0) if relu else r


def _tc_fused(panels, invd, x, w, b2d, relu):
    n, k = x.shape
    d = w.shape[1]
    np_ = len(panels)
    return pl.pallas_call(
        functools.partial(_fused_body, relu=relu, np_=np_),
        grid=(n // BN,),
        in_specs=[pl.BlockSpec((NC, BN, DP), lambda i: (0, i, 0))
                  for _ in range(np_)] + [
            pl.BlockSpec((BN, 1), lambda i: (i, 0)),
            pl.BlockSpec((BN, k), lambda i: (i, 0)),
            pl.BlockSpec((k, d), lambda i: (0, 0)),
            pl.BlockSpec((1, d), lambda i: (0, 0)),
        ],
        out_specs=pl.BlockSpec((BN, d), lambda i: (i, 0)),
        out_shape=jax.ShapeDtypeStruct((n, d), jnp.float32),
    )(*panels, invd, x, w, b2d)


def _invd_body(dp_ref, o_ref):
    deg = (dp_ref[0, :, 0:1].astype(jnp.float32)
           + dp_ref[1, :, 0:1].astype(jnp.float32))
    o_ref[...] = 1.0 / jnp.maximum(deg, 1.0)


def _tc_invd(degp):
    return pl.pallas_call(
        _invd_body,
        grid=(N // BN,),
        in_specs=[pl.BlockSpec((NC, BN, DP), lambda i: (0, i, 0))],
        out_specs=pl.BlockSpec((BN, 1), lambda i: (i, 0)),
        out_shape=jax.ShapeDtypeStruct((N, 1), jnp.float32),
    )(degp)


def kernel(x, edge_index, Wl1, Wr1, b1, Wl2, Wr2, b2, Wl3, Wr3, b3):
    src = edge_index[0]
    dst = edge_index[1]
    src2 = jnp.concatenate([src, jnp.full((N_EXTRA,), N, jnp.int32)]).reshape(EROWS, CW)
    pad_dst = N + (jnp.arange(N_EXTRA, dtype=jnp.int32) % (N_ACC - N))
    dst2 = jnp.concatenate([dst, pad_dst]).reshape(EROWS, CW)

    ones_mat = jnp.concatenate([jnp.ones((N, DP), jnp.bfloat16),
                                jnp.zeros((T_PAD - N, DP), jnp.bfloat16)])
    zpad = jnp.zeros((T_PAD - N, DP), jnp.bfloat16)

    def t_panels(h, wl, d):
        t = _tc_mm(h, wl)
        return [jnp.concatenate([t[:, j * DP:(j + 1) * DP], zpad], axis=0)
                for j in range(d // DP)]

    p1lo, p1hi, degp = _make_scatter(3)(*t_panels(x, Wl1, 128), ones_mat,
                                        src2, dst2)
    invd = _tc_invd(degp)
    h1 = _tc_fused([p1lo, p1hi], invd, x, Wr1, b1.reshape(1, 128), True)

    p2lo, p2hi = _make_scatter(2)(*t_panels(h1, Wl2, 128), src2, dst2)
    h2 = _tc_fused([p2lo, p2hi], invd, h1, Wr2, b2.reshape(1, 128), True)

    (p3,) = _make_scatter(1)(*t_panels(h2, Wl3, 64), src2, dst2)
    return _tc_fused([p3], invd, h2, Wr3, b3.reshape(1, 64), False)
